# Initial kernel scaffold; baseline (speedup 1.0000x reference)
import jax
import jax.numpy as jnp
from jax.experimental import pallas as pl


def kernel(x, edge_index, W, b, gamma, beta):
    raise NotImplementedError("write your pallas kernel here")


# speedup vs baseline: 9.5759x; 9.5759x over previous
import functools

import jax
import jax.numpy as jnp
from jax import lax
from jax.experimental import pallas as pl
from jax.experimental.pallas import tpu as pltpu
from jax.experimental.pallas import tpu_sc as plsc

N = 10000
D = 128
L = 3
E = 320000

NC = 2
NS = 16
NW = NC * NS

NPAD = 10240
ROWS_PER_TILE = NPAD // NS
K = 80
EW = K * 128
E_PAD = NW * EW
K2 = 2 * K
N_PAD_ROWS = NPAD - N


def _sc_mesh():
    return plsc.VectorSubcoreMesh(
        core_axis_name="c", subcore_axis_name="s", num_cores=NC, num_subcores=NS
    )


@functools.cache
def _build_deg_kernel():
    return functools.partial(
        pl.kernel,
        out_type=[
            jax.ShapeDtypeStruct((NPAD,), jnp.float32),
            jax.ShapeDtypeStruct((NPAD,), jnp.float32),
        ],
        mesh=_sc_mesh(),
        scratch_types=[
            pltpu.VMEM((K, 128), jnp.int32),
            pltpu.VMEM((128,), jnp.float32),
            pltpu.VMEM((640,), jnp.float32),
            pltpu.VMEM_SHARED((NPAD,), jnp.float32),
        ],
    )(_deg_body)


def _deg_body(dst_hbm, dega_hbm, degb_hbm, dstv, onesv, zbuf, acc):
    c = lax.axis_index("c")
    s = lax.axis_index("s")
    wid = c * NS + s
    r0 = s * ROWS_PER_TILE

    for i in range(8):
        onesv[pl.ds(16 * i, 16)] = jnp.ones((16,), jnp.float32)
    for i in range(40):
        zbuf[pl.ds(16 * i, 16)] = jnp.zeros((16,), jnp.float32)
    pltpu.sync_copy(zbuf.at[pl.ds(0, ROWS_PER_TILE)], acc.at[pl.ds(r0, ROWS_PER_TILE)])
    pltpu.sync_copy(dst_hbm.at[wid], dstv)
    plsc.subcore_barrier()

    def body(j, _):
        pltpu.sync_copy(onesv, acc.at[dstv.at[j]], add=True)
        return 0

    lax.fori_loop(0, K, body, 0)
    plsc.subcore_barrier()

    @pl.when(c == 0)
    def _():
        pltpu.sync_copy(acc.at[pl.ds(r0, ROWS_PER_TILE)], dega_hbm.at[pl.ds(r0, ROWS_PER_TILE)])

    @pl.when(c == 1)
    def _():
        pltpu.sync_copy(acc.at[pl.ds(r0, ROWS_PER_TILE)], degb_hbm.at[pl.ds(r0, ROWS_PER_TILE)])


HALF = NPAD // 2
TRASH = 128
ACC_ROWS = HALF + TRASH
ZROWS_PER_TILE = ACC_ROWS // NS
XROWS_PER_TILE = HALF // NS


@functools.cache
def _build_segsum_kernel():
    return functools.partial(
        pl.kernel,
        out_type=jax.ShapeDtypeStruct((NPAD, D), jnp.float32),
        mesh=_sc_mesh(),
        scratch_types=[
            pltpu.VMEM((K2, 128), jnp.int32),
            pltpu.VMEM((K2, 128), jnp.int32),
            pltpu.VMEM((128, D), jnp.float32),
            pltpu.VMEM((128, D), jnp.float32),
            pltpu.VMEM_SHARED((ACC_ROWS, D), jnp.float32),
            pltpu.SemaphoreType.DMA,
            pltpu.SemaphoreType.DMA,
        ],
    )(_segsum_body)


def _segsum_body(hs_hbm, src_hbm, dst_hbm, zeros_hbm, out_hbm,
                 srcv, dstv, rows0, rows1, acc, sem0, sem1):
    c = lax.axis_index("c")
    s = lax.axis_index("s")
    base = c * HALF

    pltpu.sync_copy(src_hbm.at[s], srcv)
    pltpu.sync_copy(dst_hbm.at[s], dstv)
    pltpu.sync_copy(zeros_hbm.at[pl.ds(s * ZROWS_PER_TILE, ZROWS_PER_TILE)],
                    acc.at[pl.ds(s * ZROWS_PER_TILE, ZROWS_PER_TILE)])

    def redirect(j, _):
        for k in range(8):
            v = dstv[j, pl.ds(16 * k, 16)]
            local = v - base
            oob = (local < 0) | (local >= HALF)
            dstv[j, pl.ds(16 * k, 16)] = jnp.where(
                oob, HALF + (v & (TRASH - 1)), local)
        return 0

    lax.fori_loop(0, K2, redirect, 0)
    plsc.subcore_barrier()

    def body(j, _):
        pltpu.async_copy(hs_hbm.at[srcv.at[j]], rows0, sem0).wait()
        pltpu.sync_copy(rows0, acc.at[dstv.at[j]], add=True)
        return 0

    lax.fori_loop(0, K2, body, 0)
    plsc.subcore_barrier()

    pltpu.sync_copy(
        acc.at[pl.ds(s * XROWS_PER_TILE, XROWS_PER_TILE)],
        out_hbm.at[pl.ds(base + s * XROWS_PER_TILE, XROWS_PER_TILE)])


R = ROWS_PER_TILE


def _mm_scale_body(x_ref, w_ref, dega_ref, degb_ref, out_ref):
    dinv = lax.rsqrt(dega_ref[...] + degb_ref[...] + 1.0)
    out_ref[...] = jnp.dot(x_ref[...], w_ref[...],
                           preferred_element_type=jnp.float32) * dinv


def _mm_scale(x, w, dega, degb):
    return pl.pallas_call(
        _mm_scale_body,
        grid=(NPAD // R,),
        in_specs=[
            pl.BlockSpec((R, D), lambda i: (i, 0)),
            pl.BlockSpec((D, D), lambda i: (0, 0)),
            pl.BlockSpec((R, 1), lambda i: (i, 0)),
            pl.BlockSpec((R, 1), lambda i: (i, 0)),
        ],
        out_specs=pl.BlockSpec((R, D), lambda i: (i, 0)),
        out_shape=jax.ShapeDtypeStruct((NPAD, D), jnp.float32),
    )(x, w, dega, degb)


def _stats_body(acc_ref, hs_ref, dega_ref, degb_ref, b_ref,
                y_ref, s1_ref, s2_ref):
    i = pl.program_id(0)
    dinv = lax.rsqrt(dega_ref[...] + degb_ref[...] + 1.0)
    y = (acc_ref[...] + hs_ref[...]) * dinv + b_ref[...]
    row = lax.broadcasted_iota(jnp.int32, (R, 1), 0) + i * R
    y = jnp.where(row < N, y, 0.0)
    y_ref[...] = y
    p1 = jnp.sum(y, axis=0, keepdims=True)
    p2 = jnp.sum(y * y, axis=0, keepdims=True)

    @pl.when(i == 0)
    def _():
        s1_ref[...] = p1
        s2_ref[...] = p2

    @pl.when(i > 0)
    def _():
        s1_ref[...] += p1
        s2_ref[...] += p2


def _stats(acc, hs, dega, degb, b):
    return pl.pallas_call(
        _stats_body,
        grid=(NPAD // R,),
        in_specs=[
            pl.BlockSpec((R, D), lambda i: (i, 0)),
            pl.BlockSpec((R, D), lambda i: (i, 0)),
            pl.BlockSpec((R, 1), lambda i: (i, 0)),
            pl.BlockSpec((R, 1), lambda i: (i, 0)),
            pl.BlockSpec((1, D), lambda i: (0, 0)),
        ],
        out_specs=[
            pl.BlockSpec((R, D), lambda i: (i, 0)),
            pl.BlockSpec((1, D), lambda i: (0, 0)),
            pl.BlockSpec((1, D), lambda i: (0, 0)),
        ],
        out_shape=[
            jax.ShapeDtypeStruct((NPAD, D), jnp.float32),
            jax.ShapeDtypeStruct((1, D), jnp.float32),
            jax.ShapeDtypeStruct((1, D), jnp.float32),
        ],
    )(acc, hs, dega, degb, b)


def _norm_body(final, y_ref, xin_ref, s1_ref, s2_ref, g_ref, be_ref, out_ref):
    i = pl.program_id(0)
    mu = s1_ref[...] * (1.0 / N)
    var = s2_ref[...] * (1.0 / N) - mu * mu
    rstd = lax.rsqrt(var + 1e-5)
    z = g_ref[...] * (y_ref[...] - mu) * rstd + be_ref[...]
    z = jnp.maximum(z, 0.0)
    n1 = jnp.sqrt(jnp.sum(z * z, axis=1, keepdims=True))
    z = z / jnp.maximum(n1, 1e-12)
    t = xin_ref[...] + z
    if final:
        n2 = jnp.sqrt(jnp.sum(t * t, axis=1, keepdims=True))
        t = t / jnp.maximum(n2, 1e-12)
    row = lax.broadcasted_iota(jnp.int32, (R, 1), 0) + i * R
    out_ref[...] = jnp.where(row < N, t, 0.0)


def _norm(y, xin, s1, s2, g, be, final):
    return pl.pallas_call(
        functools.partial(_norm_body, final),
        grid=(NPAD // R,),
        in_specs=[
            pl.BlockSpec((R, D), lambda i: (i, 0)),
            pl.BlockSpec((R, D), lambda i: (i, 0)),
            pl.BlockSpec((1, D), lambda i: (0, 0)),
            pl.BlockSpec((1, D), lambda i: (0, 0)),
            pl.BlockSpec((1, D), lambda i: (0, 0)),
            pl.BlockSpec((1, D), lambda i: (0, 0)),
        ],
        out_specs=pl.BlockSpec((R, D), lambda i: (i, 0)),
        out_shape=jax.ShapeDtypeStruct((NPAD, D), jnp.float32),
    )(y, xin, s1, s2, g, be)


def kernel(x, edge_index, W, b, gamma, beta):
    src = edge_index[0]
    dst = edge_index[1]
    pad_idx = (N + (jnp.arange(E_PAD - E, dtype=jnp.int32) % N_PAD_ROWS))
    src_full = jnp.concatenate([src, pad_idx])
    dst_full = jnp.concatenate([dst, pad_idx])
    dst_p = dst_full.reshape(NW, K, 128)
    src_p2 = src_full.reshape(NS, K2, 128)
    dst_p2 = dst_full.reshape(NS, K2, 128)
    zeros_acc = jnp.zeros((ACC_ROWS, D), jnp.float32)
    x_p = jnp.concatenate([x, jnp.zeros((NPAD - N, D), jnp.float32)], axis=0)

    dega, degb = _build_deg_kernel()(dst_p)
    dega2 = dega.reshape(NPAD, 1)
    degb2 = degb.reshape(NPAD, 1)

    for i in range(L):
        hs = _mm_scale(x_p, W[i], dega2, degb2)
        acc = _build_segsum_kernel()(hs, src_p2, dst_p2, zeros_acc)
        y, s1, s2 = _stats(acc, hs, dega2, degb2, b[i].reshape(1, D))
        x_p = _norm(y, x_p, s1, s2, gamma[i].reshape(1, D),
                    beta[i].reshape(1, D), final=(i == L - 1))

    return x_p[:N]


# speedup vs baseline: 14.7509x; 1.5404x over previous
import functools

import jax
import jax.numpy as jnp
from jax import lax
from jax.experimental import pallas as pl
from jax.experimental.pallas import tpu as pltpu
from jax.experimental.pallas import tpu_sc as plsc

N = 10000
D = 128
L = 3
E = 320000

NC = 2
NS = 16
NW = NC * NS

NPAD = 10240
ROWS_PER_TILE = NPAD // NS
K = 80
EW = K * 128
E_PAD = NW * EW
K2 = 2 * K
N_PAD_ROWS = NPAD - N


def _sc_mesh():
    return plsc.VectorSubcoreMesh(
        core_axis_name="c", subcore_axis_name="s", num_cores=NC, num_subcores=NS
    )


@functools.cache
def _build_deg_kernel():
    return functools.partial(
        pl.kernel,
        out_type=[
            jax.ShapeDtypeStruct((NPAD,), jnp.float32),
            jax.ShapeDtypeStruct((NPAD,), jnp.float32),
        ],
        mesh=_sc_mesh(),
        scratch_types=[
            pltpu.VMEM((K, 128), jnp.int32),
            pltpu.VMEM((128,), jnp.float32),
            pltpu.VMEM((640,), jnp.float32),
            pltpu.VMEM_SHARED((NPAD,), jnp.float32),
        ],
    )(_deg_body)


def _deg_body(dst_hbm, dega_hbm, degb_hbm, dstv, onesv, zbuf, acc):
    c = lax.axis_index("c")
    s = lax.axis_index("s")
    wid = c * NS + s
    r0 = s * ROWS_PER_TILE

    for i in range(8):
        onesv[pl.ds(16 * i, 16)] = jnp.ones((16,), jnp.float32)
    for i in range(40):
        zbuf[pl.ds(16 * i, 16)] = jnp.zeros((16,), jnp.float32)
    pltpu.sync_copy(zbuf.at[pl.ds(0, ROWS_PER_TILE)], acc.at[pl.ds(r0, ROWS_PER_TILE)])
    pltpu.sync_copy(dst_hbm.at[wid], dstv)
    plsc.subcore_barrier()

    def body(j, _):
        pltpu.sync_copy(onesv, acc.at[dstv.at[j]], add=True)
        return 0

    lax.fori_loop(0, K, body, 0)
    plsc.subcore_barrier()

    @pl.when(c == 0)
    def _():
        pltpu.sync_copy(acc.at[pl.ds(r0, ROWS_PER_TILE)], dega_hbm.at[pl.ds(r0, ROWS_PER_TILE)])

    @pl.when(c == 1)
    def _():
        pltpu.sync_copy(acc.at[pl.ds(r0, ROWS_PER_TILE)], degb_hbm.at[pl.ds(r0, ROWS_PER_TILE)])


HALF = NPAD // 2
TRASH = 128
ACC_ROWS = HALF + TRASH
ZROWS_PER_TILE = ACC_ROWS // NS
XROWS_PER_TILE = HALF // NS


@functools.cache
def _build_segsum_kernel():
    return functools.partial(
        pl.kernel,
        out_type=jax.ShapeDtypeStruct((NPAD, D), jnp.float32),
        mesh=_sc_mesh(),
        scratch_types=[
            pltpu.VMEM((K2, 128), jnp.int32),
            pltpu.VMEM((K2, 128), jnp.int32),
            pltpu.VMEM((128, D), jnp.float32),
            pltpu.VMEM((128, D), jnp.float32),
            pltpu.VMEM_SHARED((ACC_ROWS, D), jnp.float32),
            pltpu.SemaphoreType.DMA,
            pltpu.SemaphoreType.DMA,
        ],
    )(_segsum_body)


def _segsum_body(hs_hbm, src_hbm, dst_hbm, zeros_hbm, out_hbm,
                 srcv, dstv, rows0, rows1, acc, sem0, sem1):
    c = lax.axis_index("c")
    s = lax.axis_index("s")
    base = c * HALF

    pltpu.sync_copy(src_hbm.at[s], srcv)
    pltpu.sync_copy(dst_hbm.at[s], dstv)
    pltpu.sync_copy(zeros_hbm.at[pl.ds(s * ZROWS_PER_TILE, ZROWS_PER_TILE)],
                    acc.at[pl.ds(s * ZROWS_PER_TILE, ZROWS_PER_TILE)])

    def redirect(j, _):
        for k in range(8):
            v = dstv[j, pl.ds(16 * k, 16)]
            local = v - base
            oob = (local < 0) | (local >= HALF)
            dstv[j, pl.ds(16 * k, 16)] = jnp.where(
                oob, HALF + (v & (TRASH - 1)), local)
        return 0

    lax.fori_loop(0, K2, redirect, 0)
    plsc.subcore_barrier()

    pltpu.async_copy(hs_hbm.at[srcv.at[0]], rows0, sem0)
    pltpu.async_copy(hs_hbm.at[srcv.at[1]], rows1, sem1)

    def body(g, _):
        j0 = 2 * g
        j1 = 2 * g + 1
        pltpu.make_async_copy(hs_hbm.at[srcv.at[j0]], rows0, sem0).wait()
        pltpu.sync_copy(rows0, acc.at[dstv.at[j0]], add=True)
        pltpu.async_copy(hs_hbm.at[srcv.at[j0 + 2]], rows0, sem0)
        pltpu.make_async_copy(hs_hbm.at[srcv.at[j1]], rows1, sem1).wait()
        pltpu.sync_copy(rows1, acc.at[dstv.at[j1]], add=True)
        pltpu.async_copy(hs_hbm.at[srcv.at[j1 + 2]], rows1, sem1)
        return 0

    lax.fori_loop(0, K2 // 2 - 1, body, 0)
    pltpu.make_async_copy(hs_hbm.at[srcv.at[K2 - 2]], rows0, sem0).wait()
    pltpu.sync_copy(rows0, acc.at[dstv.at[K2 - 2]], add=True)
    pltpu.make_async_copy(hs_hbm.at[srcv.at[K2 - 1]], rows1, sem1).wait()
    pltpu.sync_copy(rows1, acc.at[dstv.at[K2 - 1]], add=True)
    plsc.subcore_barrier()

    pltpu.sync_copy(
        acc.at[pl.ds(s * XROWS_PER_TILE, XROWS_PER_TILE)],
        out_hbm.at[pl.ds(base + s * XROWS_PER_TILE, XROWS_PER_TILE)])


R = ROWS_PER_TILE


def _mm_scale_body(x_ref, w_ref, dega_ref, degb_ref, out_ref):
    dinv = lax.rsqrt(dega_ref[...] + degb_ref[...] + 1.0)
    out_ref[...] = jnp.dot(x_ref[...], w_ref[...],
                           preferred_element_type=jnp.float32) * dinv


def _mm_scale(x, w, dega, degb):
    return pl.pallas_call(
        _mm_scale_body,
        grid=(NPAD // R,),
        in_specs=[
            pl.BlockSpec((R, D), lambda i: (i, 0)),
            pl.BlockSpec((D, D), lambda i: (0, 0)),
            pl.BlockSpec((R, 1), lambda i: (i, 0)),
            pl.BlockSpec((R, 1), lambda i: (i, 0)),
        ],
        out_specs=pl.BlockSpec((R, D), lambda i: (i, 0)),
        out_shape=jax.ShapeDtypeStruct((NPAD, D), jnp.float32),
    )(x, w, dega, degb)


def _stats_body(acc_ref, hs_ref, dega_ref, degb_ref, b_ref,
                y_ref, s1_ref, s2_ref):
    i = pl.program_id(0)
    dinv = lax.rsqrt(dega_ref[...] + degb_ref[...] + 1.0)
    y = (acc_ref[...] + hs_ref[...]) * dinv + b_ref[...]
    row = lax.broadcasted_iota(jnp.int32, (R, 1), 0) + i * R
    y = jnp.where(row < N, y, 0.0)
    y_ref[...] = y
    p1 = jnp.sum(y, axis=0, keepdims=True)
    p2 = jnp.sum(y * y, axis=0, keepdims=True)

    @pl.when(i == 0)
    def _():
        s1_ref[...] = p1
        s2_ref[...] = p2

    @pl.when(i > 0)
    def _():
        s1_ref[...] += p1
        s2_ref[...] += p2


def _stats(acc, hs, dega, degb, b):
    return pl.pallas_call(
        _stats_body,
        grid=(NPAD // R,),
        in_specs=[
            pl.BlockSpec((R, D), lambda i: (i, 0)),
            pl.BlockSpec((R, D), lambda i: (i, 0)),
            pl.BlockSpec((R, 1), lambda i: (i, 0)),
            pl.BlockSpec((R, 1), lambda i: (i, 0)),
            pl.BlockSpec((1, D), lambda i: (0, 0)),
        ],
        out_specs=[
            pl.BlockSpec((R, D), lambda i: (i, 0)),
            pl.BlockSpec((1, D), lambda i: (0, 0)),
            pl.BlockSpec((1, D), lambda i: (0, 0)),
        ],
        out_shape=[
            jax.ShapeDtypeStruct((NPAD, D), jnp.float32),
            jax.ShapeDtypeStruct((1, D), jnp.float32),
            jax.ShapeDtypeStruct((1, D), jnp.float32),
        ],
    )(acc, hs, dega, degb, b)


def _norm_body(final, y_ref, xin_ref, s1_ref, s2_ref, g_ref, be_ref, out_ref):
    i = pl.program_id(0)
    mu = s1_ref[...] * (1.0 / N)
    var = s2_ref[...] * (1.0 / N) - mu * mu
    rstd = lax.rsqrt(var + 1e-5)
    z = g_ref[...] * (y_ref[...] - mu) * rstd + be_ref[...]
    z = jnp.maximum(z, 0.0)
    n1 = jnp.sqrt(jnp.sum(z * z, axis=1, keepdims=True))
    z = z / jnp.maximum(n1, 1e-12)
    t = xin_ref[...] + z
    if final:
        n2 = jnp.sqrt(jnp.sum(t * t, axis=1, keepdims=True))
        t = t / jnp.maximum(n2, 1e-12)
    row = lax.broadcasted_iota(jnp.int32, (R, 1), 0) + i * R
    out_ref[...] = jnp.where(row < N, t, 0.0)


def _norm(y, xin, s1, s2, g, be, final):
    return pl.pallas_call(
        functools.partial(_norm_body, final),
        grid=(NPAD // R,),
        in_specs=[
            pl.BlockSpec((R, D), lambda i: (i, 0)),
            pl.BlockSpec((R, D), lambda i: (i, 0)),
            pl.BlockSpec((1, D), lambda i: (0, 0)),
            pl.BlockSpec((1, D), lambda i: (0, 0)),
            pl.BlockSpec((1, D), lambda i: (0, 0)),
            pl.BlockSpec((1, D), lambda i: (0, 0)),
        ],
        out_specs=pl.BlockSpec((R, D), lambda i: (i, 0)),
        out_shape=jax.ShapeDtypeStruct((NPAD, D), jnp.float32),
    )(y, xin, s1, s2, g, be)


def kernel(x, edge_index, W, b, gamma, beta):
    src = edge_index[0]
    dst = edge_index[1]
    pad_idx = (N + (jnp.arange(E_PAD - E, dtype=jnp.int32) % N_PAD_ROWS))
    src_full = jnp.concatenate([src, pad_idx])
    dst_full = jnp.concatenate([dst, pad_idx])
    dst_p = dst_full.reshape(NW, K, 128)
    src_p2 = src_full.reshape(NS, K2, 128)
    dst_p2 = dst_full.reshape(NS, K2, 128)
    zeros_acc = jnp.zeros((ACC_ROWS, D), jnp.float32)
    x_p = jnp.concatenate([x, jnp.zeros((NPAD - N, D), jnp.float32)], axis=0)

    dega, degb = _build_deg_kernel()(dst_p)
    dega2 = dega.reshape(NPAD, 1)
    degb2 = degb.reshape(NPAD, 1)

    for i in range(L):
        hs = _mm_scale(x_p, W[i], dega2, degb2)
        acc = _build_segsum_kernel()(hs, src_p2, dst_p2, zeros_acc)
        y, s1, s2 = _stats(acc, hs, dega2, degb2, b[i].reshape(1, D))
        x_p = _norm(y, x_p, s1, s2, gamma[i].reshape(1, D),
                    beta[i].reshape(1, D), final=(i == L - 1))

    return x_p[:N]


# speedup vs baseline: 19.4451x; 1.3182x over previous
import functools

import jax
import jax.numpy as jnp
from jax import lax
from jax.experimental import pallas as pl
from jax.experimental.pallas import tpu as pltpu
from jax.experimental.pallas import tpu_sc as plsc

N = 10000
D = 128
HD = D // 2
L = 3
E = 320000

NC = 2
NS = 16
NW = NC * NS

NPAD = 10240
ROWS_PER_TILE = NPAD // NS
K = 80
EW = K * 128
E_PAD = NW * EW
K2 = 2 * K
N_PAD_ROWS = NPAD - N


def _sc_mesh():
    return plsc.VectorSubcoreMesh(
        core_axis_name="c", subcore_axis_name="s", num_cores=NC, num_subcores=NS
    )


@functools.cache
def _build_deg_kernel():
    return functools.partial(
        pl.kernel,
        out_type=[
            jax.ShapeDtypeStruct((NPAD,), jnp.float32),
            jax.ShapeDtypeStruct((NPAD,), jnp.float32),
        ],
        mesh=_sc_mesh(),
        scratch_types=[
            pltpu.VMEM((K, 128), jnp.int32),
            pltpu.VMEM((128,), jnp.float32),
            pltpu.VMEM((640,), jnp.float32),
            pltpu.VMEM_SHARED((NPAD,), jnp.float32),
        ],
    )(_deg_body)


def _deg_body(dst_hbm, dega_hbm, degb_hbm, dstv, onesv, zbuf, acc):
    c = lax.axis_index("c")
    s = lax.axis_index("s")
    wid = c * NS + s
    r0 = s * ROWS_PER_TILE

    for i in range(8):
        onesv[pl.ds(16 * i, 16)] = jnp.ones((16,), jnp.float32)
    for i in range(40):
        zbuf[pl.ds(16 * i, 16)] = jnp.zeros((16,), jnp.float32)
    pltpu.sync_copy(zbuf.at[pl.ds(0, ROWS_PER_TILE)], acc.at[pl.ds(r0, ROWS_PER_TILE)])
    pltpu.sync_copy(dst_hbm.at[wid], dstv)
    plsc.subcore_barrier()

    def body(j, _):
        pltpu.sync_copy(onesv, acc.at[dstv.at[j]], add=True)
        return 0

    lax.fori_loop(0, K, body, 0)
    plsc.subcore_barrier()

    @pl.when(c == 0)
    def _():
        pltpu.sync_copy(acc.at[pl.ds(r0, ROWS_PER_TILE)], dega_hbm.at[pl.ds(r0, ROWS_PER_TILE)])

    @pl.when(c == 1)
    def _():
        pltpu.sync_copy(acc.at[pl.ds(r0, ROWS_PER_TILE)], degb_hbm.at[pl.ds(r0, ROWS_PER_TILE)])


@functools.cache
def _build_segsum_kernel():
    return functools.partial(
        pl.kernel,
        out_type=[
            jax.ShapeDtypeStruct((NPAD, HD), jnp.float32),
            jax.ShapeDtypeStruct((NPAD, HD), jnp.float32),
        ],
        mesh=_sc_mesh(),
        compiler_params=pltpu.CompilerParams(use_tc_tiling_on_sc=False),
        scratch_types=[
            pltpu.VMEM((K2, 128), jnp.int32),
            pltpu.VMEM((K2, 128), jnp.int32),
            pltpu.VMEM((128, HD), jnp.float32),
            pltpu.VMEM((128, HD), jnp.float32),
            pltpu.VMEM_SHARED((NPAD, HD), jnp.float32),
            pltpu.SemaphoreType.DMA,
            pltpu.SemaphoreType.DMA,
        ],
    )(_segsum_body)


def _segsum_body(hs2_hbm, src_hbm, dst_hbm, zeros_hbm, outa_hbm, outb_hbm,
                 srcv, dstv, rows0, rows1, acc, sem0, sem1):
    c = lax.axis_index("c")
    s = lax.axis_index("s")
    r0 = s * ROWS_PER_TILE

    pltpu.sync_copy(src_hbm.at[s], srcv)
    pltpu.sync_copy(dst_hbm.at[s], dstv)
    pltpu.sync_copy(zeros_hbm.at[pl.ds(r0, ROWS_PER_TILE)],
                    acc.at[pl.ds(r0, ROWS_PER_TILE)])

    def rewrite(j, _):
        for k in range(8):
            v = srcv[j, pl.ds(16 * k, 16)]
            srcv[j, pl.ds(16 * k, 16)] = 2 * v + c
        return 0

    lax.fori_loop(0, K2, rewrite, 0)
    plsc.subcore_barrier()

    pltpu.async_copy(hs2_hbm.at[srcv.at[0]], rows0, sem0)
    pltpu.async_copy(hs2_hbm.at[srcv.at[1]], rows1, sem1)

    def body(g, _):
        j0 = 2 * g
        j1 = 2 * g + 1
        pltpu.make_async_copy(hs2_hbm.at[srcv.at[j0]], rows0, sem0).wait()
        pltpu.sync_copy(rows0, acc.at[dstv.at[j0]], add=True)
        pltpu.async_copy(hs2_hbm.at[srcv.at[j0 + 2]], rows0, sem0)
        pltpu.make_async_copy(hs2_hbm.at[srcv.at[j1]], rows1, sem1).wait()
        pltpu.sync_copy(rows1, acc.at[dstv.at[j1]], add=True)
        pltpu.async_copy(hs2_hbm.at[srcv.at[j1 + 2]], rows1, sem1)
        return 0

    lax.fori_loop(0, K2 // 2 - 1, body, 0)
    pltpu.make_async_copy(hs2_hbm.at[srcv.at[K2 - 2]], rows0, sem0).wait()
    pltpu.sync_copy(rows0, acc.at[dstv.at[K2 - 2]], add=True)
    pltpu.make_async_copy(hs2_hbm.at[srcv.at[K2 - 1]], rows1, sem1).wait()
    pltpu.sync_copy(rows1, acc.at[dstv.at[K2 - 1]], add=True)
    plsc.subcore_barrier()

    @pl.when(c == 0)
    def _():
        pltpu.sync_copy(acc.at[pl.ds(r0, ROWS_PER_TILE)],
                        outa_hbm.at[pl.ds(r0, ROWS_PER_TILE)])

    @pl.when(c == 1)
    def _():
        pltpu.sync_copy(acc.at[pl.ds(r0, ROWS_PER_TILE)],
                        outb_hbm.at[pl.ds(r0, ROWS_PER_TILE)])


R = NPAD // NS


def _mm_scale_body(x_ref, w_ref, dega_ref, degb_ref, out_ref):
    dinv = lax.rsqrt(dega_ref[...] + degb_ref[...] + 1.0)
    out_ref[...] = jnp.dot(x_ref[...], w_ref[...],
                           preferred_element_type=jnp.float32) * dinv


def _mm_scale(x, w, dega, degb):
    return pl.pallas_call(
        _mm_scale_body,
        grid=(NPAD // R,),
        in_specs=[
            pl.BlockSpec((R, D), lambda i: (i, 0)),
            pl.BlockSpec((D, D), lambda i: (0, 0)),
            pl.BlockSpec((R, 1), lambda i: (i, 0)),
            pl.BlockSpec((R, 1), lambda i: (i, 0)),
        ],
        out_specs=pl.BlockSpec((R, D), lambda i: (i, 0)),
        out_shape=jax.ShapeDtypeStruct((NPAD, D), jnp.float32),
    )(x, w, dega, degb)


def _stats_body(acca_ref, accb_ref, hs_ref, dega_ref, degb_ref, b_ref,
                y_ref, s1_ref, s2_ref):
    i = pl.program_id(0)
    dinv = lax.rsqrt(dega_ref[...] + degb_ref[...] + 1.0)
    accf = jnp.concatenate([acca_ref[...], accb_ref[...]], axis=1)
    y = (accf + hs_ref[...]) * dinv + b_ref[...]
    row = lax.broadcasted_iota(jnp.int32, (R, 1), 0) + i * R
    y = jnp.where(row < N, y, 0.0)
    y_ref[...] = y
    p1 = jnp.sum(y, axis=0, keepdims=True)
    p2 = jnp.sum(y * y, axis=0, keepdims=True)

    @pl.when(i == 0)
    def _():
        s1_ref[...] = p1
        s2_ref[...] = p2

    @pl.when(i > 0)
    def _():
        s1_ref[...] += p1
        s2_ref[...] += p2


def _stats(acca, accb, hs, dega, degb, b):
    return pl.pallas_call(
        _stats_body,
        grid=(NPAD // R,),
        in_specs=[
            pl.BlockSpec((R, HD), lambda i: (i, 0)),
            pl.BlockSpec((R, HD), lambda i: (i, 0)),
            pl.BlockSpec((R, D), lambda i: (i, 0)),
            pl.BlockSpec((R, 1), lambda i: (i, 0)),
            pl.BlockSpec((R, 1), lambda i: (i, 0)),
            pl.BlockSpec((1, D), lambda i: (0, 0)),
        ],
        out_specs=[
            pl.BlockSpec((R, D), lambda i: (i, 0)),
            pl.BlockSpec((1, D), lambda i: (0, 0)),
            pl.BlockSpec((1, D), lambda i: (0, 0)),
        ],
        out_shape=[
            jax.ShapeDtypeStruct((NPAD, D), jnp.float32),
            jax.ShapeDtypeStruct((1, D), jnp.float32),
            jax.ShapeDtypeStruct((1, D), jnp.float32),
        ],
    )(acca, accb, hs, dega, degb, b)


def _norm_body(final, y_ref, xin_ref, s1_ref, s2_ref, g_ref, be_ref, out_ref):
    i = pl.program_id(0)
    mu = s1_ref[...] * (1.0 / N)
    var = s2_ref[...] * (1.0 / N) - mu * mu
    rstd = lax.rsqrt(var + 1e-5)
    z = g_ref[...] * (y_ref[...] - mu) * rstd + be_ref[...]
    z = jnp.maximum(z, 0.0)
    n1 = jnp.sqrt(jnp.sum(z * z, axis=1, keepdims=True))
    z = z / jnp.maximum(n1, 1e-12)
    t = xin_ref[...] + z
    if final:
        n2 = jnp.sqrt(jnp.sum(t * t, axis=1, keepdims=True))
        t = t / jnp.maximum(n2, 1e-12)
    row = lax.broadcasted_iota(jnp.int32, (R, 1), 0) + i * R
    out_ref[...] = jnp.where(row < N, t, 0.0)


def _norm(y, xin, s1, s2, g, be, final):
    return pl.pallas_call(
        functools.partial(_norm_body, final),
        grid=(NPAD // R,),
        in_specs=[
            pl.BlockSpec((R, D), lambda i: (i, 0)),
            pl.BlockSpec((R, D), lambda i: (i, 0)),
            pl.BlockSpec((1, D), lambda i: (0, 0)),
            pl.BlockSpec((1, D), lambda i: (0, 0)),
            pl.BlockSpec((1, D), lambda i: (0, 0)),
            pl.BlockSpec((1, D), lambda i: (0, 0)),
        ],
        out_specs=pl.BlockSpec((R, D), lambda i: (i, 0)),
        out_shape=jax.ShapeDtypeStruct((NPAD, D), jnp.float32),
    )(y, xin, s1, s2, g, be)


def kernel(x, edge_index, W, b, gamma, beta):
    src = edge_index[0]
    dst = edge_index[1]
    pad_idx = (N + (jnp.arange(E_PAD - E, dtype=jnp.int32) % N_PAD_ROWS))
    src_full = jnp.concatenate([src, pad_idx])
    dst_full = jnp.concatenate([dst, pad_idx])
    dst_p = dst_full.reshape(NW, K, 128)
    src_p2 = src_full.reshape(NS, K2, 128)
    dst_p2 = dst_full.reshape(NS, K2, 128)
    zeros_acc = jnp.zeros((NPAD, HD), jnp.float32)
    x_p = jnp.concatenate([x, jnp.zeros((NPAD - N, D), jnp.float32)], axis=0)

    dega, degb = _build_deg_kernel()(dst_p)
    dega2 = dega.reshape(NPAD, 1)
    degb2 = degb.reshape(NPAD, 1)

    for i in range(L):
        hs = _mm_scale(x_p, W[i], dega2, degb2)
        hs2 = hs.reshape(2 * NPAD, HD)
        acca, accb = _build_segsum_kernel()(hs2, src_p2, dst_p2, zeros_acc)
        y, s1, s2 = _stats(acca, accb, hs, dega2, degb2, b[i].reshape(1, D))
        x_p = _norm(y, x_p, s1, s2, gamma[i].reshape(1, D),
                    beta[i].reshape(1, D), final=(i == L - 1))

    return x_p[:N]


# speedup vs baseline: 23.1947x; 1.1928x over previous
import functools

import jax
import jax.numpy as jnp
from jax import lax
from jax.experimental import pallas as pl
from jax.experimental.pallas import tpu as pltpu
from jax.experimental.pallas import tpu_sc as plsc

N = 10000
D = 128
HD = D // 2
L = 3
E = 320000

NC = 2
NS = 16
NW = NC * NS

NPAD = 10240
ROWS_PER_TILE = NPAD // NS
K = 80
EW = K * 128
E_PAD = NW * EW
K2 = 2 * K
N_PAD_ROWS = NPAD - N


def _sc_mesh():
    return plsc.VectorSubcoreMesh(
        core_axis_name="c", subcore_axis_name="s", num_cores=NC, num_subcores=NS
    )


@functools.cache
def _build_deg_kernel():
    return functools.partial(
        pl.kernel,
        out_type=[
            jax.ShapeDtypeStruct((NPAD,), jnp.float32),
            jax.ShapeDtypeStruct((NPAD,), jnp.float32),
        ],
        mesh=_sc_mesh(),
        scratch_types=[
            pltpu.VMEM((K, 128), jnp.int32),
            pltpu.VMEM((128,), jnp.float32),
            pltpu.VMEM((640,), jnp.float32),
            pltpu.VMEM_SHARED((NPAD,), jnp.float32),
        ],
    )(_deg_body)


def _deg_body(dst_hbm, dega_hbm, degb_hbm, dstv, onesv, zbuf, acc):
    c = lax.axis_index("c")
    s = lax.axis_index("s")
    wid = c * NS + s
    r0 = s * ROWS_PER_TILE

    for i in range(8):
        onesv[pl.ds(16 * i, 16)] = jnp.ones((16,), jnp.float32)
    for i in range(40):
        zbuf[pl.ds(16 * i, 16)] = jnp.zeros((16,), jnp.float32)
    pltpu.sync_copy(zbuf.at[pl.ds(0, ROWS_PER_TILE)], acc.at[pl.ds(r0, ROWS_PER_TILE)])
    pltpu.sync_copy(dst_hbm.at[wid], dstv)
    plsc.subcore_barrier()

    def body(j, _):
        pltpu.sync_copy(onesv, acc.at[dstv.at[j]], add=True)
        return 0

    lax.fori_loop(0, K, body, 0)
    plsc.subcore_barrier()

    @pl.when(c == 0)
    def _():
        pltpu.sync_copy(acc.at[pl.ds(r0, ROWS_PER_TILE)], dega_hbm.at[pl.ds(r0, ROWS_PER_TILE)])

    @pl.when(c == 1)
    def _():
        pltpu.sync_copy(acc.at[pl.ds(r0, ROWS_PER_TILE)], degb_hbm.at[pl.ds(r0, ROWS_PER_TILE)])


@functools.cache
def _build_segsum_kernel():
    return functools.partial(
        pl.kernel,
        out_type=[
            jax.ShapeDtypeStruct((NPAD, HD), jnp.float32),
            jax.ShapeDtypeStruct((NPAD, HD), jnp.float32),
        ],
        mesh=_sc_mesh(),
        compiler_params=pltpu.CompilerParams(use_tc_tiling_on_sc=False),
        scratch_types=[
            pltpu.VMEM((K2, 128), jnp.int32),
            pltpu.VMEM((K2, 128), jnp.int32),
            pltpu.VMEM((128, HD), jnp.float32),
            pltpu.VMEM((128, HD), jnp.float32),
            pltpu.VMEM((128, HD), jnp.float32),
            pltpu.VMEM((128, HD), jnp.float32),
            pltpu.SemaphoreType.DMA,
            pltpu.SemaphoreType.DMA,
            pltpu.SemaphoreType.DMA,
            pltpu.SemaphoreType.DMA,
            pltpu.VMEM_SHARED((NPAD, HD), jnp.float32),
        ],
    )(_segsum_body)


def _segsum_body(hs2_hbm, src_hbm, dst_hbm, zeros_hbm, outa_hbm, outb_hbm,
                 srcv, dstv, rows0, rows1, rows2, rows3,
                 sem0, sem1, sem2, sem3, acc):
    c = lax.axis_index("c")
    s = lax.axis_index("s")
    r0 = s * ROWS_PER_TILE

    pltpu.sync_copy(src_hbm.at[s], srcv)
    pltpu.sync_copy(dst_hbm.at[s], dstv)
    pltpu.sync_copy(zeros_hbm.at[pl.ds(r0, ROWS_PER_TILE)],
                    acc.at[pl.ds(r0, ROWS_PER_TILE)])

    def rewrite(j, _):
        for k in range(8):
            v = srcv[j, pl.ds(16 * k, 16)]
            srcv[j, pl.ds(16 * k, 16)] = 2 * v + c
        return 0

    lax.fori_loop(0, K2, rewrite, 0)
    plsc.subcore_barrier()

    bufs = (rows0, rows1, rows2, rows3)
    sems = (sem0, sem1, sem2, sem3)
    for b in range(4):
        pltpu.async_copy(hs2_hbm.at[srcv.at[b]], bufs[b], sems[b])

    def body(g, _):
        for b in range(4):
            j = 4 * g + b
            pltpu.make_async_copy(hs2_hbm.at[srcv.at[j]], bufs[b], sems[b]).wait()
            pltpu.sync_copy(bufs[b], acc.at[dstv.at[j]], add=True)
            pltpu.async_copy(hs2_hbm.at[srcv.at[j + 4]], bufs[b], sems[b])
        return 0

    lax.fori_loop(0, K2 // 4 - 1, body, 0)
    for b in range(4):
        j = K2 - 4 + b
        pltpu.make_async_copy(hs2_hbm.at[srcv.at[j]], bufs[b], sems[b]).wait()
        pltpu.sync_copy(bufs[b], acc.at[dstv.at[j]], add=True)
    plsc.subcore_barrier()

    @pl.when(c == 0)
    def _():
        pltpu.sync_copy(acc.at[pl.ds(r0, ROWS_PER_TILE)],
                        outa_hbm.at[pl.ds(r0, ROWS_PER_TILE)])

    @pl.when(c == 1)
    def _():
        pltpu.sync_copy(acc.at[pl.ds(r0, ROWS_PER_TILE)],
                        outb_hbm.at[pl.ds(r0, ROWS_PER_TILE)])


R = NPAD // NS


def _mm_scale_body(x_ref, w_ref, dega_ref, degb_ref, out_ref):
    dinv = lax.rsqrt(dega_ref[...] + degb_ref[...] + 1.0)
    out_ref[...] = jnp.dot(x_ref[...], w_ref[...],
                           preferred_element_type=jnp.float32) * dinv


def _mm_scale(x, w, dega, degb):
    return pl.pallas_call(
        _mm_scale_body,
        grid=(NPAD // R,),
        in_specs=[
            pl.BlockSpec((R, D), lambda i: (i, 0)),
            pl.BlockSpec((D, D), lambda i: (0, 0)),
            pl.BlockSpec((R, 1), lambda i: (i, 0)),
            pl.BlockSpec((R, 1), lambda i: (i, 0)),
        ],
        out_specs=pl.BlockSpec((R, D), lambda i: (i, 0)),
        out_shape=jax.ShapeDtypeStruct((NPAD, D), jnp.float32),
    )(x, w, dega, degb)


def _stats_body(acca_ref, accb_ref, hs_ref, dega_ref, degb_ref, b_ref,
                y_ref, s1_ref, s2_ref):
    i = pl.program_id(0)
    dinv = lax.rsqrt(dega_ref[...] + degb_ref[...] + 1.0)
    accf = jnp.concatenate([acca_ref[...], accb_ref[...]], axis=1)
    y = (accf + hs_ref[...]) * dinv + b_ref[...]
    row = lax.broadcasted_iota(jnp.int32, (R, 1), 0) + i * R
    y = jnp.where(row < N, y, 0.0)
    y_ref[...] = y
    p1 = jnp.sum(y, axis=0, keepdims=True)
    p2 = jnp.sum(y * y, axis=0, keepdims=True)

    @pl.when(i == 0)
    def _():
        s1_ref[...] = p1
        s2_ref[...] = p2

    @pl.when(i > 0)
    def _():
        s1_ref[...] += p1
        s2_ref[...] += p2


def _stats(acca, accb, hs, dega, degb, b):
    return pl.pallas_call(
        _stats_body,
        grid=(NPAD // R,),
        in_specs=[
            pl.BlockSpec((R, HD), lambda i: (i, 0)),
            pl.BlockSpec((R, HD), lambda i: (i, 0)),
            pl.BlockSpec((R, D), lambda i: (i, 0)),
            pl.BlockSpec((R, 1), lambda i: (i, 0)),
            pl.BlockSpec((R, 1), lambda i: (i, 0)),
            pl.BlockSpec((1, D), lambda i: (0, 0)),
        ],
        out_specs=[
            pl.BlockSpec((R, D), lambda i: (i, 0)),
            pl.BlockSpec((1, D), lambda i: (0, 0)),
            pl.BlockSpec((1, D), lambda i: (0, 0)),
        ],
        out_shape=[
            jax.ShapeDtypeStruct((NPAD, D), jnp.float32),
            jax.ShapeDtypeStruct((1, D), jnp.float32),
            jax.ShapeDtypeStruct((1, D), jnp.float32),
        ],
    )(acca, accb, hs, dega, degb, b)


def _norm_body(final, y_ref, xin_ref, s1_ref, s2_ref, g_ref, be_ref, out_ref):
    i = pl.program_id(0)
    mu = s1_ref[...] * (1.0 / N)
    var = s2_ref[...] * (1.0 / N) - mu * mu
    rstd = lax.rsqrt(var + 1e-5)
    z = g_ref[...] * (y_ref[...] - mu) * rstd + be_ref[...]
    z = jnp.maximum(z, 0.0)
    n1 = jnp.sqrt(jnp.sum(z * z, axis=1, keepdims=True))
    z = z / jnp.maximum(n1, 1e-12)
    t = xin_ref[...] + z
    if final:
        n2 = jnp.sqrt(jnp.sum(t * t, axis=1, keepdims=True))
        t = t / jnp.maximum(n2, 1e-12)
    row = lax.broadcasted_iota(jnp.int32, (R, 1), 0) + i * R
    out_ref[...] = jnp.where(row < N, t, 0.0)


def _norm(y, xin, s1, s2, g, be, final):
    return pl.pallas_call(
        functools.partial(_norm_body, final),
        grid=(NPAD // R,),
        in_specs=[
            pl.BlockSpec((R, D), lambda i: (i, 0)),
            pl.BlockSpec((R, D), lambda i: (i, 0)),
            pl.BlockSpec((1, D), lambda i: (0, 0)),
            pl.BlockSpec((1, D), lambda i: (0, 0)),
            pl.BlockSpec((1, D), lambda i: (0, 0)),
            pl.BlockSpec((1, D), lambda i: (0, 0)),
        ],
        out_specs=pl.BlockSpec((R, D), lambda i: (i, 0)),
        out_shape=jax.ShapeDtypeStruct((NPAD, D), jnp.float32),
    )(y, xin, s1, s2, g, be)


def kernel(x, edge_index, W, b, gamma, beta):
    src = edge_index[0]
    dst = edge_index[1]
    pad_idx = (N + (jnp.arange(E_PAD - E, dtype=jnp.int32) % N_PAD_ROWS))
    src_full = jnp.concatenate([src, pad_idx])
    dst_full = jnp.concatenate([dst, pad_idx])
    dst_p = dst_full.reshape(NW, K, 128)
    src_p2 = src_full.reshape(NS, K2, 128)
    dst_p2 = dst_full.reshape(NS, K2, 128)
    zeros_acc = jnp.zeros((NPAD, HD), jnp.float32)
    x_p = jnp.concatenate([x, jnp.zeros((NPAD - N, D), jnp.float32)], axis=0)

    dega, degb = _build_deg_kernel()(dst_p)
    dega2 = dega.reshape(NPAD, 1)
    degb2 = degb.reshape(NPAD, 1)

    for i in range(L):
        hs = _mm_scale(x_p, W[i], dega2, degb2)
        hs2 = hs.reshape(2 * NPAD, HD)
        acca, accb = _build_segsum_kernel()(hs2, src_p2, dst_p2, zeros_acc)
        y, s1, s2 = _stats(acca, accb, hs, dega2, degb2, b[i].reshape(1, D))
        x_p = _norm(y, x_p, s1, s2, gamma[i].reshape(1, D),
                    beta[i].reshape(1, D), final=(i == L - 1))

    return x_p[:N]


# speedup vs baseline: 24.1853x; 1.0427x over previous
import functools

import jax
import jax.numpy as jnp
from jax import lax
from jax.experimental import pallas as pl
from jax.experimental.pallas import tpu as pltpu
from jax.experimental.pallas import tpu_sc as plsc

N = 10000
D = 128
HD = D // 2
L = 3
E = 320000

NC = 2
NS = 16
NW = NC * NS

NPAD = 10240
ROWS_PER_TILE = NPAD // NS
K = 80
EW = K * 128
E_PAD = NW * EW
K2 = 2 * K
N_PAD_ROWS = NPAD - N


def _sc_mesh():
    return plsc.VectorSubcoreMesh(
        core_axis_name="c", subcore_axis_name="s", num_cores=NC, num_subcores=NS
    )


@functools.cache
def _build_deg_kernel():
    return functools.partial(
        pl.kernel,
        out_type=[
            jax.ShapeDtypeStruct((NPAD,), jnp.float32),
            jax.ShapeDtypeStruct((NPAD,), jnp.float32),
        ],
        mesh=_sc_mesh(),
        scratch_types=[
            pltpu.VMEM((K, 128), jnp.int32),
            pltpu.VMEM((128,), jnp.float32),
            pltpu.VMEM((640,), jnp.float32),
            pltpu.VMEM_SHARED((NPAD,), jnp.float32),
        ],
    )(_deg_body)


def _deg_body(dst_hbm, dega_hbm, degb_hbm, dstv, onesv, zbuf, acc):
    c = lax.axis_index("c")
    s = lax.axis_index("s")
    wid = c * NS + s
    r0 = s * ROWS_PER_TILE

    for i in range(8):
        onesv[pl.ds(16 * i, 16)] = jnp.ones((16,), jnp.float32)
    for i in range(40):
        zbuf[pl.ds(16 * i, 16)] = jnp.zeros((16,), jnp.float32)
    pltpu.sync_copy(zbuf.at[pl.ds(0, ROWS_PER_TILE)], acc.at[pl.ds(r0, ROWS_PER_TILE)])
    pltpu.sync_copy(dst_hbm.at[wid], dstv)
    plsc.subcore_barrier()

    def body(j, _):
        pltpu.sync_copy(onesv, acc.at[dstv.at[j]], add=True)
        return 0

    lax.fori_loop(0, K, body, 0)
    plsc.subcore_barrier()

    @pl.when(c == 0)
    def _():
        pltpu.sync_copy(acc.at[pl.ds(r0, ROWS_PER_TILE)], dega_hbm.at[pl.ds(r0, ROWS_PER_TILE)])

    @pl.when(c == 1)
    def _():
        pltpu.sync_copy(acc.at[pl.ds(r0, ROWS_PER_TILE)], degb_hbm.at[pl.ds(r0, ROWS_PER_TILE)])


@functools.cache
def _build_segsum_kernel():
    return functools.partial(
        pl.kernel,
        out_type=[
            jax.ShapeDtypeStruct((NPAD, HD), jnp.float32),
            jax.ShapeDtypeStruct((NPAD, HD), jnp.float32),
        ],
        mesh=_sc_mesh(),
        compiler_params=pltpu.CompilerParams(use_tc_tiling_on_sc=False),
        scratch_types=[
            pltpu.VMEM((K2, 128), jnp.int32),
            pltpu.VMEM((K2, 128), jnp.int32),
            pltpu.VMEM((128, HD), jnp.float32),
            pltpu.VMEM((128, HD), jnp.float32),
            pltpu.VMEM((128, HD), jnp.float32),
            pltpu.VMEM((128, HD), jnp.float32),
            pltpu.SemaphoreType.DMA,
            pltpu.SemaphoreType.DMA,
            pltpu.SemaphoreType.DMA,
            pltpu.SemaphoreType.DMA,
            pltpu.VMEM_SHARED((NPAD, HD), jnp.float32),
        ],
    )(_segsum_body)


def _segsum_body(hs2_hbm, src_hbm, dst_hbm, zeros_hbm, outa_hbm, outb_hbm,
                 srcv, dstv, rows0, rows1, rows2, rows3,
                 sem0, sem1, sem2, sem3, acc):
    c = lax.axis_index("c")
    s = lax.axis_index("s")
    r0 = s * ROWS_PER_TILE

    pltpu.sync_copy(src_hbm.at[s], srcv)
    pltpu.sync_copy(dst_hbm.at[s], dstv)
    pltpu.sync_copy(zeros_hbm.at[pl.ds(r0, ROWS_PER_TILE)],
                    acc.at[pl.ds(r0, ROWS_PER_TILE)])

    def rewrite(j, _):
        for k in range(8):
            v = srcv[j, pl.ds(16 * k, 16)]
            srcv[j, pl.ds(16 * k, 16)] = 2 * v + c
        return 0

    lax.fori_loop(0, K2, rewrite, 0)
    plsc.subcore_barrier()

    bufs = (rows0, rows1, rows2, rows3)
    sems = (sem0, sem1, sem2, sem3)
    for b in range(4):
        pltpu.async_copy(hs2_hbm.at[srcv.at[b]], bufs[b], sems[b])

    def body(g, _):
        for b in range(4):
            j = 4 * g + b
            pltpu.make_async_copy(hs2_hbm.at[srcv.at[j]], bufs[b], sems[b]).wait()
            pltpu.sync_copy(bufs[b], acc.at[dstv.at[j]], add=True)
            pltpu.async_copy(hs2_hbm.at[srcv.at[j + 4]], bufs[b], sems[b])
        return 0

    lax.fori_loop(0, K2 // 4 - 1, body, 0)
    for b in range(4):
        j = K2 - 4 + b
        pltpu.make_async_copy(hs2_hbm.at[srcv.at[j]], bufs[b], sems[b]).wait()
        pltpu.sync_copy(bufs[b], acc.at[dstv.at[j]], add=True)
    plsc.subcore_barrier()

    @pl.when(c == 0)
    def _():
        pltpu.sync_copy(acc.at[pl.ds(r0, ROWS_PER_TILE)],
                        outa_hbm.at[pl.ds(r0, ROWS_PER_TILE)])

    @pl.when(c == 1)
    def _():
        pltpu.sync_copy(acc.at[pl.ds(r0, ROWS_PER_TILE)],
                        outb_hbm.at[pl.ds(r0, ROWS_PER_TILE)])


R = NPAD // NS


def _mm_scale_body(x_ref, w_ref, dega_ref, degb_ref, out_ref):
    dinv = lax.rsqrt(dega_ref[...] + degb_ref[...] + 1.0)
    out_ref[...] = jnp.dot(x_ref[...], w_ref[...],
                           preferred_element_type=jnp.float32) * dinv


def _mm_scale(x, w, dega, degb):
    return pl.pallas_call(
        _mm_scale_body,
        grid=(NPAD // R,),
        in_specs=[
            pl.BlockSpec((R, D), lambda i: (i, 0)),
            pl.BlockSpec((D, D), lambda i: (0, 0)),
            pl.BlockSpec((R, 1), lambda i: (i, 0)),
            pl.BlockSpec((R, 1), lambda i: (i, 0)),
        ],
        out_specs=pl.BlockSpec((R, D), lambda i: (i, 0)),
        out_shape=jax.ShapeDtypeStruct((NPAD, D), jnp.float32),
    )(x, w, dega, degb)


def _stats_body(acca_ref, accb_ref, hs_ref, dega_ref, degb_ref, b_ref,
                y_ref, s1_ref, s2_ref):
    i = pl.program_id(0)
    dinv = lax.rsqrt(dega_ref[...] + degb_ref[...] + 1.0)
    accf = jnp.concatenate([acca_ref[...], accb_ref[...]], axis=1)
    y = (accf + hs_ref[...]) * dinv + b_ref[...]
    row = lax.broadcasted_iota(jnp.int32, (R, 1), 0) + i * R
    y = jnp.where(row < N, y, 0.0)
    y_ref[...] = y
    p1 = jnp.sum(y, axis=0, keepdims=True)
    p2 = jnp.sum(y * y, axis=0, keepdims=True)

    @pl.when(i == 0)
    def _():
        s1_ref[...] = p1
        s2_ref[...] = p2

    @pl.when(i > 0)
    def _():
        s1_ref[...] += p1
        s2_ref[...] += p2


def _stats(acca, accb, hs, dega, degb, b):
    return pl.pallas_call(
        _stats_body,
        grid=(NPAD // R,),
        in_specs=[
            pl.BlockSpec((R, HD), lambda i: (i, 0)),
            pl.BlockSpec((R, HD), lambda i: (i, 0)),
            pl.BlockSpec((R, D), lambda i: (i, 0)),
            pl.BlockSpec((R, 1), lambda i: (i, 0)),
            pl.BlockSpec((R, 1), lambda i: (i, 0)),
            pl.BlockSpec((1, D), lambda i: (0, 0)),
        ],
        out_specs=[
            pl.BlockSpec((R, D), lambda i: (i, 0)),
            pl.BlockSpec((1, D), lambda i: (0, 0)),
            pl.BlockSpec((1, D), lambda i: (0, 0)),
        ],
        out_shape=[
            jax.ShapeDtypeStruct((NPAD, D), jnp.float32),
            jax.ShapeDtypeStruct((1, D), jnp.float32),
            jax.ShapeDtypeStruct((1, D), jnp.float32),
        ],
    )(acca, accb, hs, dega, degb, b)


def _norm_body(final, y_ref, xin_ref, s1_ref, s2_ref, g_ref, be_ref, out_ref):
    i = pl.program_id(0)
    mu = s1_ref[...] * (1.0 / N)
    var = s2_ref[...] * (1.0 / N) - mu * mu
    rstd = lax.rsqrt(var + 1e-5)
    z = g_ref[...] * (y_ref[...] - mu) * rstd + be_ref[...]
    z = jnp.maximum(z, 0.0)
    n1 = jnp.sqrt(jnp.sum(z * z, axis=1, keepdims=True))
    z = z / jnp.maximum(n1, 1e-12)
    t = xin_ref[...] + z
    if final:
        n2 = jnp.sqrt(jnp.sum(t * t, axis=1, keepdims=True))
        t = t / jnp.maximum(n2, 1e-12)
    row = lax.broadcasted_iota(jnp.int32, (R, 1), 0) + i * R
    out_ref[...] = jnp.where(row < N, t, 0.0)


def _norm(y, xin, s1, s2, g, be, final):
    return pl.pallas_call(
        functools.partial(_norm_body, final),
        grid=(NPAD // R,),
        in_specs=[
            pl.BlockSpec((R, D), lambda i: (i, 0)),
            pl.BlockSpec((R, D), lambda i: (i, 0)),
            pl.BlockSpec((1, D), lambda i: (0, 0)),
            pl.BlockSpec((1, D), lambda i: (0, 0)),
            pl.BlockSpec((1, D), lambda i: (0, 0)),
            pl.BlockSpec((1, D), lambda i: (0, 0)),
        ],
        out_specs=pl.BlockSpec((R, D), lambda i: (i, 0)),
        out_shape=jax.ShapeDtypeStruct((NPAD, D), jnp.float32),
    )(y, xin, s1, s2, g, be)


def _norm_mm_body(y_ref, xin_ref, s1_ref, s2_ref, g_ref, be_ref, w_ref,
                  dega_ref, degb_ref, xout_ref, hs_ref):
    i = pl.program_id(0)
    mu = s1_ref[...] * (1.0 / N)
    var = s2_ref[...] * (1.0 / N) - mu * mu
    rstd = lax.rsqrt(var + 1e-5)
    z = g_ref[...] * (y_ref[...] - mu) * rstd + be_ref[...]
    z = jnp.maximum(z, 0.0)
    n1 = jnp.sqrt(jnp.sum(z * z, axis=1, keepdims=True))
    z = z / jnp.maximum(n1, 1e-12)
    t = xin_ref[...] + z
    row = lax.broadcasted_iota(jnp.int32, (R, 1), 0) + i * R
    t = jnp.where(row < N, t, 0.0)
    xout_ref[...] = t
    dinv = lax.rsqrt(dega_ref[...] + degb_ref[...] + 1.0)
    hs_ref[...] = jnp.dot(t, w_ref[...],
                          preferred_element_type=jnp.float32) * dinv


def _norm_mm(y, xin, s1, s2, g, be, w, dega, degb):
    return pl.pallas_call(
        _norm_mm_body,
        grid=(NPAD // R,),
        in_specs=[
            pl.BlockSpec((R, D), lambda i: (i, 0)),
            pl.BlockSpec((R, D), lambda i: (i, 0)),
            pl.BlockSpec((1, D), lambda i: (0, 0)),
            pl.BlockSpec((1, D), lambda i: (0, 0)),
            pl.BlockSpec((1, D), lambda i: (0, 0)),
            pl.BlockSpec((1, D), lambda i: (0, 0)),
            pl.BlockSpec((D, D), lambda i: (0, 0)),
            pl.BlockSpec((R, 1), lambda i: (i, 0)),
            pl.BlockSpec((R, 1), lambda i: (i, 0)),
        ],
        out_specs=[
            pl.BlockSpec((R, D), lambda i: (i, 0)),
            pl.BlockSpec((R, D), lambda i: (i, 0)),
        ],
        out_shape=[
            jax.ShapeDtypeStruct((NPAD, D), jnp.float32),
            jax.ShapeDtypeStruct((NPAD, D), jnp.float32),
        ],
    )(y, xin, s1, s2, g, be, w, dega, degb)


def kernel(x, edge_index, W, b, gamma, beta):
    src = edge_index[0]
    dst = edge_index[1]
    pad_idx = (N + (jnp.arange(E_PAD - E, dtype=jnp.int32) % N_PAD_ROWS))
    src_full = jnp.concatenate([src, pad_idx])
    dst_full = jnp.concatenate([dst, pad_idx])
    dst_p = dst_full.reshape(NW, K, 128)
    src_p2 = src_full.reshape(NS, K2, 128)
    dst_p2 = dst_full.reshape(NS, K2, 128)
    zeros_acc = jnp.zeros((NPAD, HD), jnp.float32)
    x_p = jnp.concatenate([x, jnp.zeros((NPAD - N, D), jnp.float32)], axis=0)

    dega, degb = _build_deg_kernel()(dst_p)
    dega2 = dega.reshape(NPAD, 1)
    degb2 = degb.reshape(NPAD, 1)

    hs = _mm_scale(x_p, W[0], dega2, degb2)
    for i in range(L):
        hs2 = hs.reshape(2 * NPAD, HD)
        acca, accb = _build_segsum_kernel()(hs2, src_p2, dst_p2, zeros_acc)
        y, s1, s2 = _stats(acca, accb, hs, dega2, degb2, b[i].reshape(1, D))
        if i < L - 1:
            x_p, hs = _norm_mm(y, x_p, s1, s2, gamma[i].reshape(1, D),
                               beta[i].reshape(1, D), W[i + 1], dega2, degb2)
        else:
            x_p = _norm(y, x_p, s1, s2, gamma[i].reshape(1, D),
                        beta[i].reshape(1, D), final=True)

    return x_p[:N]


# speedup vs baseline: 24.3978x; 1.0088x over previous
import functools

import jax
import jax.numpy as jnp
from jax import lax
from jax.experimental import pallas as pl
from jax.experimental.pallas import tpu as pltpu
from jax.experimental.pallas import tpu_sc as plsc

N = 10000
D = 128
HD = D // 2
L = 3
E = 320000

NC = 2
NS = 16
NW = NC * NS

NPAD = 10240
ROWS_PER_TILE = NPAD // NS
K = 80
EW = K * 128
E_PAD = NW * EW
K2 = 2 * K
N_PAD_ROWS = NPAD - N


def _sc_mesh():
    return plsc.VectorSubcoreMesh(
        core_axis_name="c", subcore_axis_name="s", num_cores=NC, num_subcores=NS
    )


@functools.cache
def _build_deg_kernel():
    return functools.partial(
        pl.kernel,
        out_type=[
            jax.ShapeDtypeStruct((NPAD,), jnp.float32),
            jax.ShapeDtypeStruct((NPAD,), jnp.float32),
        ],
        mesh=_sc_mesh(),
        scratch_types=[
            pltpu.VMEM((K, 128), jnp.int32),
            pltpu.VMEM((128,), jnp.float32),
            pltpu.VMEM((640,), jnp.float32),
            pltpu.VMEM_SHARED((NPAD,), jnp.float32),
        ],
    )(_deg_body)


def _deg_body(dst_hbm, dega_hbm, degb_hbm, dstv, onesv, zbuf, acc):
    c = lax.axis_index("c")
    s = lax.axis_index("s")
    wid = c * NS + s
    r0 = s * ROWS_PER_TILE

    for i in range(8):
        onesv[pl.ds(16 * i, 16)] = jnp.ones((16,), jnp.float32)
    for i in range(40):
        zbuf[pl.ds(16 * i, 16)] = jnp.zeros((16,), jnp.float32)
    pltpu.sync_copy(zbuf.at[pl.ds(0, ROWS_PER_TILE)], acc.at[pl.ds(r0, ROWS_PER_TILE)])
    pltpu.sync_copy(dst_hbm.at[wid], dstv)
    plsc.subcore_barrier()

    def body(j, _):
        pltpu.sync_copy(onesv, acc.at[dstv.at[j]], add=True)
        return 0

    lax.fori_loop(0, K, body, 0)
    plsc.subcore_barrier()

    @pl.when(c == 0)
    def _():
        pltpu.sync_copy(acc.at[pl.ds(r0, ROWS_PER_TILE)], dega_hbm.at[pl.ds(r0, ROWS_PER_TILE)])

    @pl.when(c == 1)
    def _():
        pltpu.sync_copy(acc.at[pl.ds(r0, ROWS_PER_TILE)], degb_hbm.at[pl.ds(r0, ROWS_PER_TILE)])


@functools.cache
def _build_segsum_kernel():
    return functools.partial(
        pl.kernel,
        out_type=[
            jax.ShapeDtypeStruct((NPAD, HD), jnp.float32),
            jax.ShapeDtypeStruct((NPAD, HD), jnp.float32),
        ],
        mesh=_sc_mesh(),
        compiler_params=pltpu.CompilerParams(use_tc_tiling_on_sc=False),
        scratch_types=[
            pltpu.VMEM((K2, 128), jnp.int32),
            pltpu.VMEM((K2, 128), jnp.int32),
            pltpu.VMEM((128, HD), jnp.float32),
            pltpu.VMEM((128, HD), jnp.float32),
            pltpu.VMEM((128, HD), jnp.float32),
            pltpu.VMEM((128, HD), jnp.float32),
            pltpu.SemaphoreType.DMA,
            pltpu.SemaphoreType.DMA,
            pltpu.SemaphoreType.DMA,
            pltpu.SemaphoreType.DMA,
            pltpu.VMEM_SHARED((NPAD, HD), jnp.float32),
        ],
    )(_segsum_body)


def _segsum_body(hs2_hbm, src_hbm, dst_hbm, zeros_hbm, outa_hbm, outb_hbm,
                 srcv, dstv, rows0, rows1, rows2, rows3,
                 sem0, sem1, sem2, sem3, acc):
    c = lax.axis_index("c")
    s = lax.axis_index("s")
    r0 = s * ROWS_PER_TILE

    pltpu.sync_copy(src_hbm.at[s], srcv)
    pltpu.sync_copy(dst_hbm.at[s], dstv)
    pltpu.sync_copy(zeros_hbm.at[pl.ds(r0, ROWS_PER_TILE)],
                    acc.at[pl.ds(r0, ROWS_PER_TILE)])

    def rewrite(j, _):
        for k in range(8):
            v = srcv[j, pl.ds(16 * k, 16)]
            srcv[j, pl.ds(16 * k, 16)] = 2 * v + c
        return 0

    lax.fori_loop(0, K2, rewrite, 0)
    plsc.subcore_barrier()

    bufs = (rows0, rows1, rows2, rows3)
    sems = (sem0, sem1, sem2, sem3)
    for b in range(4):
        pltpu.async_copy(hs2_hbm.at[srcv.at[b]], bufs[b], sems[b])

    def body(g, _):
        for b in range(4):
            j = 4 * g + b
            pltpu.make_async_copy(hs2_hbm.at[srcv.at[j]], bufs[b], sems[b]).wait()
            pltpu.sync_copy(bufs[b], acc.at[dstv.at[j]], add=True)
            pltpu.async_copy(hs2_hbm.at[srcv.at[j + 4]], bufs[b], sems[b])
        return 0

    lax.fori_loop(0, K2 // 4 - 1, body, 0)
    for b in range(4):
        j = K2 - 4 + b
        pltpu.make_async_copy(hs2_hbm.at[srcv.at[j]], bufs[b], sems[b]).wait()
        pltpu.sync_copy(bufs[b], acc.at[dstv.at[j]], add=True)
    plsc.subcore_barrier()

    @pl.when(c == 0)
    def _():
        pltpu.sync_copy(acc.at[pl.ds(r0, ROWS_PER_TILE)],
                        outa_hbm.at[pl.ds(r0, ROWS_PER_TILE)])

    @pl.when(c == 1)
    def _():
        pltpu.sync_copy(acc.at[pl.ds(r0, ROWS_PER_TILE)],
                        outb_hbm.at[pl.ds(r0, ROWS_PER_TILE)])


R = NPAD // NS


def _mm_scale_body(x_ref, w_ref, dega_ref, degb_ref, out_ref):
    dinv = lax.rsqrt(dega_ref[...] + degb_ref[...] + 1.0)
    out_ref[...] = jnp.dot(x_ref[...], w_ref[...],
                           preferred_element_type=jnp.float32) * dinv


def _mm_scale(x, w, dega, degb):
    return pl.pallas_call(
        _mm_scale_body,
        grid=(NPAD // R,),
        in_specs=[
            pl.BlockSpec((R, D), lambda i: (i, 0)),
            pl.BlockSpec((D, D), lambda i: (0, 0)),
            pl.BlockSpec((R, 1), lambda i: (i, 0)),
            pl.BlockSpec((R, 1), lambda i: (i, 0)),
        ],
        out_specs=pl.BlockSpec((R, D), lambda i: (i, 0)),
        out_shape=jax.ShapeDtypeStruct((NPAD, D), jnp.float32),
    )(x, w, dega, degb)


def _stats_body(acca_ref, accb_ref, hs_ref, dega_ref, degb_ref, b_ref,
                y_ref, s1_ref, s2_ref):
    i = pl.program_id(0)
    dinv = lax.rsqrt(dega_ref[...] + degb_ref[...] + 1.0)
    accf = jnp.concatenate([acca_ref[...], accb_ref[...]], axis=1)
    y = (accf + hs_ref[...]) * dinv + b_ref[...]
    row = lax.broadcasted_iota(jnp.int32, (R, 1), 0) + i * R
    y = jnp.where(row < N, y, 0.0)
    y_ref[...] = y
    p1 = jnp.sum(y, axis=0, keepdims=True)
    p2 = jnp.sum(y * y, axis=0, keepdims=True)

    @pl.when(i == 0)
    def _():
        s1_ref[...] = p1
        s2_ref[...] = p2

    @pl.when(i > 0)
    def _():
        s1_ref[...] += p1
        s2_ref[...] += p2


def _stats(acca, accb, hs, dega, degb, b):
    return pl.pallas_call(
        _stats_body,
        grid=(NPAD // R,),
        in_specs=[
            pl.BlockSpec((R, HD), lambda i: (i, 0)),
            pl.BlockSpec((R, HD), lambda i: (i, 0)),
            pl.BlockSpec((R, D), lambda i: (i, 0)),
            pl.BlockSpec((R, 1), lambda i: (i, 0)),
            pl.BlockSpec((R, 1), lambda i: (i, 0)),
            pl.BlockSpec((1, D), lambda i: (0, 0)),
        ],
        out_specs=[
            pl.BlockSpec((R, D), lambda i: (i, 0)),
            pl.BlockSpec((1, D), lambda i: (0, 0)),
            pl.BlockSpec((1, D), lambda i: (0, 0)),
        ],
        out_shape=[
            jax.ShapeDtypeStruct((NPAD, D), jnp.float32),
            jax.ShapeDtypeStruct((1, D), jnp.float32),
            jax.ShapeDtypeStruct((1, D), jnp.float32),
        ],
    )(acca, accb, hs, dega, degb, b)


def _norm_body(final, y_ref, xin_ref, s1_ref, s2_ref, g_ref, be_ref, out_ref):
    i = pl.program_id(0)
    mu = s1_ref[...] * (1.0 / N)
    var = s2_ref[...] * (1.0 / N) - mu * mu
    rstd = lax.rsqrt(var + 1e-5)
    z = g_ref[...] * (y_ref[...] - mu) * rstd + be_ref[...]
    z = jnp.maximum(z, 0.0)
    n1 = jnp.sqrt(jnp.sum(z * z, axis=1, keepdims=True))
    z = z / jnp.maximum(n1, 1e-12)
    t = xin_ref[...] + z
    if final:
        n2 = jnp.sqrt(jnp.sum(t * t, axis=1, keepdims=True))
        t = t / jnp.maximum(n2, 1e-12)
    row = lax.broadcasted_iota(jnp.int32, (R, 1), 0) + i * R
    out_ref[...] = jnp.where(row < N, t, 0.0)


def _norm(y, xin, s1, s2, g, be, final):
    return pl.pallas_call(
        functools.partial(_norm_body, final),
        grid=(NPAD // R,),
        in_specs=[
            pl.BlockSpec((R, D), lambda i: (i, 0)),
            pl.BlockSpec((R, D), lambda i: (i, 0)),
            pl.BlockSpec((1, D), lambda i: (0, 0)),
            pl.BlockSpec((1, D), lambda i: (0, 0)),
            pl.BlockSpec((1, D), lambda i: (0, 0)),
            pl.BlockSpec((1, D), lambda i: (0, 0)),
        ],
        out_specs=pl.BlockSpec((R, D), lambda i: (i, 0)),
        out_shape=jax.ShapeDtypeStruct((NPAD, D), jnp.float32),
    )(y, xin, s1, s2, g, be)


def _post_body_mm(acca_ref, accb_ref, hs_ref, dega_ref, degb_ref, b_ref,
                  xin_ref, g_ref, be_ref, w_ref, xout_ref, hs2_ref,
                  ys_ref, s1_ref, s2_ref):
    _post_core("mm", acca_ref, accb_ref, hs_ref, dega_ref, degb_ref, b_ref,
               xin_ref, g_ref, be_ref, w_ref, xout_ref, hs2_ref,
               ys_ref, s1_ref, s2_ref)


def _post_body_final(acca_ref, accb_ref, hs_ref, dega_ref, degb_ref, b_ref,
                     xin_ref, g_ref, be_ref, w_ref, xout_ref,
                     ys_ref, s1_ref, s2_ref):
    _post_core("final", acca_ref, accb_ref, hs_ref, dega_ref, degb_ref, b_ref,
               xin_ref, g_ref, be_ref, w_ref, xout_ref, None,
               ys_ref, s1_ref, s2_ref)


def _post_core(mode, acca_ref, accb_ref, hs_ref, dega_ref, degb_ref, b_ref,
               xin_ref, g_ref, be_ref, w_ref, xout_ref, hs2_ref,
               ys_ref, s1_ref, s2_ref):
    j = pl.program_id(0)
    i = pl.program_id(1)
    dinv = lax.rsqrt(dega_ref[...] + degb_ref[...] + 1.0)
    row = lax.broadcasted_iota(jnp.int32, (R, 1), 0) + i * R

    @pl.when(j == 0)
    def _():
        accf = jnp.concatenate([acca_ref[...], accb_ref[...]], axis=1)
        y = (accf + hs_ref[...]) * dinv + b_ref[...]
        y = jnp.where(row < N, y, 0.0)
        ys_ref[pl.ds(i * R, R), :] = y
        p1 = jnp.sum(y, axis=0, keepdims=True)
        p2 = jnp.sum(y * y, axis=0, keepdims=True)

        @pl.when(i == 0)
        def _():
            s1_ref[...] = p1
            s2_ref[...] = p2

        @pl.when(i > 0)
        def _():
            s1_ref[...] += p1
            s2_ref[...] += p2

    @pl.when(j == 1)
    def _():
        mu = s1_ref[...] * (1.0 / N)
        var = s2_ref[...] * (1.0 / N) - mu * mu
        rstd = lax.rsqrt(var + 1e-5)
        z = g_ref[...] * (ys_ref[pl.ds(i * R, R), :] - mu) * rstd + be_ref[...]
        z = jnp.maximum(z, 0.0)
        n1 = jnp.sqrt(jnp.sum(z * z, axis=1, keepdims=True))
        z = z / jnp.maximum(n1, 1e-12)
        t = xin_ref[...] + z
        if mode == "final":
            n2 = jnp.sqrt(jnp.sum(t * t, axis=1, keepdims=True))
            t = t / jnp.maximum(n2, 1e-12)
        t = jnp.where(row < N, t, 0.0)
        xout_ref[...] = t
        if mode == "mm":
            hs2_ref[...] = jnp.dot(t, w_ref[...],
                                   preferred_element_type=jnp.float32) * dinv


def _post(mode, acca, accb, hs, dega, degb, b, xin, g, be, w):
    z16 = lambda j, i: (0, 0)
    p0 = lambda j, i: ((1 - j) * i, 0)
    p1 = lambda j, i: (j * i, 0)
    both = lambda j, i: (i, 0)
    out_shapes = [jax.ShapeDtypeStruct((NPAD, D), jnp.float32)]
    out_specs = [pl.BlockSpec((R, D), p1)]
    if mode == "mm":
        out_shapes.append(jax.ShapeDtypeStruct((NPAD, D), jnp.float32))
        out_specs.append(pl.BlockSpec((R, D), p1))
    res = pl.pallas_call(
        _post_body_mm if mode == "mm" else _post_body_final,
        grid=(2, NS),
        in_specs=[
            pl.BlockSpec((R, HD), p0),
            pl.BlockSpec((R, HD), p0),
            pl.BlockSpec((R, D), p0),
            pl.BlockSpec((R, 1), both),
            pl.BlockSpec((R, 1), both),
            pl.BlockSpec((1, D), z16),
            pl.BlockSpec((R, D), p1),
            pl.BlockSpec((1, D), z16),
            pl.BlockSpec((1, D), z16),
            pl.BlockSpec((D, D), z16),
        ],
        out_specs=out_specs,
        out_shape=out_shapes,
        scratch_shapes=[
            pltpu.VMEM((NPAD, D), jnp.float32),
            pltpu.VMEM((1, D), jnp.float32),
            pltpu.VMEM((1, D), jnp.float32),
        ],
    )(acca, accb, hs, dega, degb, b, xin, g, be, w)
    if mode == "mm":
        return res[0], res[1]
    return res[0]


def _norm_mm_body(y_ref, xin_ref, s1_ref, s2_ref, g_ref, be_ref, w_ref,
                  dega_ref, degb_ref, xout_ref, hs_ref):
    i = pl.program_id(0)
    mu = s1_ref[...] * (1.0 / N)
    var = s2_ref[...] * (1.0 / N) - mu * mu
    rstd = lax.rsqrt(var + 1e-5)
    z = g_ref[...] * (y_ref[...] - mu) * rstd + be_ref[...]
    z = jnp.maximum(z, 0.0)
    n1 = jnp.sqrt(jnp.sum(z * z, axis=1, keepdims=True))
    z = z / jnp.maximum(n1, 1e-12)
    t = xin_ref[...] + z
    row = lax.broadcasted_iota(jnp.int32, (R, 1), 0) + i * R
    t = jnp.where(row < N, t, 0.0)
    xout_ref[...] = t
    dinv = lax.rsqrt(dega_ref[...] + degb_ref[...] + 1.0)
    hs_ref[...] = jnp.dot(t, w_ref[...],
                          preferred_element_type=jnp.float32) * dinv


def _norm_mm(y, xin, s1, s2, g, be, w, dega, degb):
    return pl.pallas_call(
        _norm_mm_body,
        grid=(NPAD // R,),
        in_specs=[
            pl.BlockSpec((R, D), lambda i: (i, 0)),
            pl.BlockSpec((R, D), lambda i: (i, 0)),
            pl.BlockSpec((1, D), lambda i: (0, 0)),
            pl.BlockSpec((1, D), lambda i: (0, 0)),
            pl.BlockSpec((1, D), lambda i: (0, 0)),
            pl.BlockSpec((1, D), lambda i: (0, 0)),
            pl.BlockSpec((D, D), lambda i: (0, 0)),
            pl.BlockSpec((R, 1), lambda i: (i, 0)),
            pl.BlockSpec((R, 1), lambda i: (i, 0)),
        ],
        out_specs=[
            pl.BlockSpec((R, D), lambda i: (i, 0)),
            pl.BlockSpec((R, D), lambda i: (i, 0)),
        ],
        out_shape=[
            jax.ShapeDtypeStruct((NPAD, D), jnp.float32),
            jax.ShapeDtypeStruct((NPAD, D), jnp.float32),
        ],
    )(y, xin, s1, s2, g, be, w, dega, degb)


def kernel(x, edge_index, W, b, gamma, beta):
    src = edge_index[0]
    dst = edge_index[1]
    pad_idx = (N + (jnp.arange(E_PAD - E, dtype=jnp.int32) % N_PAD_ROWS))
    src_full = jnp.concatenate([src, pad_idx])
    dst_full = jnp.concatenate([dst, pad_idx])
    dst_p = dst_full.reshape(NW, K, 128)
    src_p2 = src_full.reshape(NS, K2, 128)
    dst_p2 = dst_full.reshape(NS, K2, 128)
    zeros_acc = jnp.zeros((NPAD, HD), jnp.float32)
    x_p = jnp.concatenate([x, jnp.zeros((NPAD - N, D), jnp.float32)], axis=0)

    dega, degb = _build_deg_kernel()(dst_p)
    dega2 = dega.reshape(NPAD, 1)
    degb2 = degb.reshape(NPAD, 1)

    hs = _mm_scale(x_p, W[0], dega2, degb2)
    for i in range(L):
        hs2 = hs.reshape(2 * NPAD, HD)
        acca, accb = _build_segsum_kernel()(hs2, src_p2, dst_p2, zeros_acc)
        if i < L - 1:
            x_p, hs = _post("mm", acca, accb, hs, dega2, degb2,
                            b[i].reshape(1, D), x_p, gamma[i].reshape(1, D),
                            beta[i].reshape(1, D), W[i + 1])
        else:
            x_p = _post("final", acca, accb, hs, dega2, degb2,
                        b[i].reshape(1, D), x_p, gamma[i].reshape(1, D),
                        beta[i].reshape(1, D), W[i])

    return x_p[:N]


# speedup vs baseline: 24.4195x; 1.0009x over previous
import functools

import jax
import jax.numpy as jnp
from jax import lax
from jax.experimental import pallas as pl
from jax.experimental.pallas import tpu as pltpu
from jax.experimental.pallas import tpu_sc as plsc

N = 10000
D = 128
HD = D // 2
L = 3
E = 320000

NC = 2
NS = 16
NW = NC * NS

NPAD = 10240
ROWS_PER_TILE = NPAD // NS
K = 80
EW = K * 128
E_PAD = NW * EW
K2 = 2 * K
N_PAD_ROWS = NPAD - N


def _sc_mesh():
    return plsc.VectorSubcoreMesh(
        core_axis_name="c", subcore_axis_name="s", num_cores=NC, num_subcores=NS
    )


@functools.cache
def _build_deg_kernel():
    return functools.partial(
        pl.kernel,
        out_type=[
            jax.ShapeDtypeStruct((NPAD,), jnp.float32),
            jax.ShapeDtypeStruct((NPAD,), jnp.float32),
        ],
        mesh=_sc_mesh(),
        scratch_types=[
            pltpu.VMEM((K, 128), jnp.int32),
            pltpu.VMEM((128,), jnp.float32),
            pltpu.VMEM((640,), jnp.float32),
            pltpu.VMEM_SHARED((NPAD,), jnp.float32),
        ],
    )(_deg_body)


def _deg_body(dst_hbm, dega_hbm, degb_hbm, dstv, onesv, zbuf, acc):
    c = lax.axis_index("c")
    s = lax.axis_index("s")
    wid = c * NS + s
    r0 = s * ROWS_PER_TILE

    for i in range(8):
        onesv[pl.ds(16 * i, 16)] = jnp.ones((16,), jnp.float32)
    for i in range(40):
        zbuf[pl.ds(16 * i, 16)] = jnp.zeros((16,), jnp.float32)
    pltpu.sync_copy(zbuf.at[pl.ds(0, ROWS_PER_TILE)], acc.at[pl.ds(r0, ROWS_PER_TILE)])
    pltpu.sync_copy(dst_hbm.at[wid], dstv)
    plsc.subcore_barrier()

    def body(j, _):
        pltpu.sync_copy(onesv, acc.at[dstv.at[j]], add=True)
        return 0

    lax.fori_loop(0, K, body, 0)
    plsc.subcore_barrier()

    @pl.when(c == 0)
    def _():
        pltpu.sync_copy(acc.at[pl.ds(r0, ROWS_PER_TILE)], dega_hbm.at[pl.ds(r0, ROWS_PER_TILE)])

    @pl.when(c == 1)
    def _():
        pltpu.sync_copy(acc.at[pl.ds(r0, ROWS_PER_TILE)], degb_hbm.at[pl.ds(r0, ROWS_PER_TILE)])


@functools.cache
def _build_segsum_kernel():
    return functools.partial(
        pl.kernel,
        out_type=[
            jax.ShapeDtypeStruct((NPAD, HD), jnp.float32),
            jax.ShapeDtypeStruct((NPAD, HD), jnp.float32),
        ],
        mesh=_sc_mesh(),
        compiler_params=pltpu.CompilerParams(use_tc_tiling_on_sc=False),
        scratch_types=[
            pltpu.VMEM((K2, 128), jnp.int32),
            pltpu.VMEM((K2, 128), jnp.int32),
            pltpu.VMEM((128, HD), jnp.float32),
            pltpu.VMEM((128, HD), jnp.float32),
            pltpu.VMEM((128, HD), jnp.float32),
            pltpu.VMEM((128, HD), jnp.float32),
            pltpu.SemaphoreType.DMA,
            pltpu.SemaphoreType.DMA,
            pltpu.SemaphoreType.DMA,
            pltpu.SemaphoreType.DMA,
            pltpu.VMEM_SHARED((NPAD, HD), jnp.float32),
        ],
    )(_segsum_body)


def _segsum_body(hs2_hbm, src_hbm, dst_hbm, zeros_hbm, outa_hbm, outb_hbm,
                 srcv, dstv, rows0, rows1, rows2, rows3,
                 sem0, sem1, sem2, sem3, acc):
    c = lax.axis_index("c")
    s = lax.axis_index("s")
    r0 = s * ROWS_PER_TILE

    pltpu.sync_copy(src_hbm.at[s], srcv)
    pltpu.sync_copy(dst_hbm.at[s], dstv)
    pltpu.sync_copy(zeros_hbm.at[pl.ds(r0, ROWS_PER_TILE)],
                    acc.at[pl.ds(r0, ROWS_PER_TILE)])

    def rewrite(j, _):
        for k in range(8):
            v = srcv[j, pl.ds(16 * k, 16)]
            srcv[j, pl.ds(16 * k, 16)] = 2 * v + c
        return 0

    lax.fori_loop(0, K2, rewrite, 0)
    plsc.subcore_barrier()

    bufs = (rows0, rows1, rows2, rows3)
    sems = (sem0, sem1, sem2, sem3)
    for b in range(4):
        pltpu.async_copy(hs2_hbm.at[srcv.at[b]], bufs[b], sems[b])

    def body(g, _):
        for b in range(4):
            j = 4 * g + b
            pltpu.make_async_copy(hs2_hbm.at[srcv.at[j]], bufs[b], sems[b]).wait()
            pltpu.sync_copy(bufs[b], acc.at[dstv.at[j]], add=True)
            pltpu.async_copy(hs2_hbm.at[srcv.at[j + 4]], bufs[b], sems[b])
        return 0

    lax.fori_loop(0, K2 // 4 - 1, body, 0)
    for b in range(4):
        j = K2 - 4 + b
        pltpu.make_async_copy(hs2_hbm.at[srcv.at[j]], bufs[b], sems[b]).wait()
        pltpu.sync_copy(bufs[b], acc.at[dstv.at[j]], add=True)
    plsc.subcore_barrier()

    @pl.when(c == 0)
    def _():
        pltpu.sync_copy(acc.at[pl.ds(r0, ROWS_PER_TILE)],
                        outa_hbm.at[pl.ds(r0, ROWS_PER_TILE)])

    @pl.when(c == 1)
    def _():
        pltpu.sync_copy(acc.at[pl.ds(r0, ROWS_PER_TILE)],
                        outb_hbm.at[pl.ds(r0, ROWS_PER_TILE)])


R = NPAD // NS


def _mm_scale_body(x_ref, w_ref, dega_ref, degb_ref, out_ref):
    dinv = lax.rsqrt(dega_ref[...] + degb_ref[...] + 1.0)
    out_ref[...] = jnp.dot(x_ref[...], w_ref[...],
                           preferred_element_type=jnp.float32) * dinv


def _mm_scale(x, w, dega, degb):
    return pl.pallas_call(
        _mm_scale_body,
        grid=(NPAD // R,),
        in_specs=[
            pl.BlockSpec((R, D), lambda i: (i, 0)),
            pl.BlockSpec((D, D), lambda i: (0, 0)),
            pl.BlockSpec((R, 1), lambda i: (i, 0)),
            pl.BlockSpec((R, 1), lambda i: (i, 0)),
        ],
        out_specs=pl.BlockSpec((R, D), lambda i: (i, 0)),
        out_shape=jax.ShapeDtypeStruct((NPAD, D), jnp.float32),
    )(x, w, dega, degb)


def _post_body_mm(acca_ref, accb_ref, hs_ref, dega_ref, degb_ref, b_ref,
                  xin_ref, g_ref, be_ref, w_ref, xout_ref, hs2_ref,
                  ys_ref, s1_ref, s2_ref):
    _post_core("mm", acca_ref, accb_ref, hs_ref, dega_ref, degb_ref, b_ref,
               xin_ref, g_ref, be_ref, w_ref, xout_ref, hs2_ref,
               ys_ref, s1_ref, s2_ref)


def _post_body_final(acca_ref, accb_ref, hs_ref, dega_ref, degb_ref, b_ref,
                     xin_ref, g_ref, be_ref, w_ref, xout_ref,
                     ys_ref, s1_ref, s2_ref):
    _post_core("final", acca_ref, accb_ref, hs_ref, dega_ref, degb_ref, b_ref,
               xin_ref, g_ref, be_ref, w_ref, xout_ref, None,
               ys_ref, s1_ref, s2_ref)


def _post_core(mode, acca_ref, accb_ref, hs_ref, dega_ref, degb_ref, b_ref,
               xin_ref, g_ref, be_ref, w_ref, xout_ref, hs2_ref,
               ys_ref, s1_ref, s2_ref):
    j = pl.program_id(0)
    i = pl.program_id(1)
    dinv = lax.rsqrt(dega_ref[...] + degb_ref[...] + 1.0)
    row = lax.broadcasted_iota(jnp.int32, (R, 1), 0) + i * R

    @pl.when(j == 0)
    def _():
        accf = jnp.concatenate([acca_ref[...], accb_ref[...]], axis=1)
        y = (accf + hs_ref[...]) * dinv + b_ref[...]
        y = jnp.where(row < N, y, 0.0)
        ys_ref[pl.ds(i * R, R), :] = y
        p1 = jnp.sum(y, axis=0, keepdims=True)
        p2 = jnp.sum(y * y, axis=0, keepdims=True)

        @pl.when(i == 0)
        def _():
            s1_ref[...] = p1
            s2_ref[...] = p2

        @pl.when(i > 0)
        def _():
            s1_ref[...] += p1
            s2_ref[...] += p2

    @pl.when(j == 1)
    def _():
        mu = s1_ref[...] * (1.0 / N)
        var = s2_ref[...] * (1.0 / N) - mu * mu
        rstd = lax.rsqrt(var + 1e-5)
        z = g_ref[...] * (ys_ref[pl.ds(i * R, R), :] - mu) * rstd + be_ref[...]
        z = jnp.maximum(z, 0.0)
        n1 = jnp.sqrt(jnp.sum(z * z, axis=1, keepdims=True))
        z = z / jnp.maximum(n1, 1e-12)
        t = xin_ref[...] + z
        if mode == "final":
            n2 = jnp.sqrt(jnp.sum(t * t, axis=1, keepdims=True))
            t = t / jnp.maximum(n2, 1e-12)
        t = jnp.where(row < N, t, 0.0)
        xout_ref[...] = t
        if mode == "mm":
            hs2_ref[...] = jnp.dot(t, w_ref[...],
                                   preferred_element_type=jnp.float32) * dinv


def _post(mode, acca, accb, hs, dega, degb, b, xin, g, be, w):
    z16 = lambda j, i: (0, 0)
    p0 = lambda j, i: ((1 - j) * i, 0)
    p1 = lambda j, i: (j * i, 0)
    both = lambda j, i: (i, 0)
    out_shapes = [jax.ShapeDtypeStruct((NPAD, D), jnp.float32)]
    out_specs = [pl.BlockSpec((R, D), p1)]
    if mode == "mm":
        out_shapes.append(jax.ShapeDtypeStruct((NPAD, D), jnp.float32))
        out_specs.append(pl.BlockSpec((R, D), p1))
    res = pl.pallas_call(
        _post_body_mm if mode == "mm" else _post_body_final,
        grid=(2, NS),
        in_specs=[
            pl.BlockSpec((R, HD), p0),
            pl.BlockSpec((R, HD), p0),
            pl.BlockSpec((R, D), p0),
            pl.BlockSpec((R, 1), both),
            pl.BlockSpec((R, 1), both),
            pl.BlockSpec((1, D), z16),
            pl.BlockSpec((R, D), p1),
            pl.BlockSpec((1, D), z16),
            pl.BlockSpec((1, D), z16),
            pl.BlockSpec((D, D), z16),
        ],
        out_specs=out_specs,
        out_shape=out_shapes,
        scratch_shapes=[
            pltpu.VMEM((NPAD, D), jnp.float32),
            pltpu.VMEM((1, D), jnp.float32),
            pltpu.VMEM((1, D), jnp.float32),
        ],
    )(acca, accb, hs, dega, degb, b, xin, g, be, w)
    if mode == "mm":
        return res[0], res[1]
    return res[0]


def kernel(x, edge_index, W, b, gamma, beta):
    src = edge_index[0]
    dst = edge_index[1]
    pad_idx = (N + (jnp.arange(E_PAD - E, dtype=jnp.int32) % N_PAD_ROWS))
    src_full = jnp.concatenate([src, pad_idx])
    dst_full = jnp.concatenate([dst, pad_idx])
    dst_p = dst_full.reshape(NW, K, 128)
    src_p2 = src_full.reshape(NS, K2, 128)
    dst_p2 = dst_full.reshape(NS, K2, 128)
    zeros_acc = jnp.zeros((NPAD, HD), jnp.float32)
    x_p = jnp.concatenate([x, jnp.zeros((NPAD - N, D), jnp.float32)], axis=0)

    dega, degb = _build_deg_kernel()(dst_p)
    dega2 = dega.reshape(NPAD, 1)
    degb2 = degb.reshape(NPAD, 1)

    hs = _mm_scale(x_p, W[0], dega2, degb2)
    for i in range(L):
        hs2 = hs.reshape(2 * NPAD, HD)
        acca, accb = _build_segsum_kernel()(hs2, src_p2, dst_p2, zeros_acc)
        if i < L - 1:
            x_p, hs = _post("mm", acca, accb, hs, dega2, degb2,
                            b[i].reshape(1, D), x_p, gamma[i].reshape(1, D),
                            beta[i].reshape(1, D), W[i + 1])
        else:
            x_p = _post("final", acca, accb, hs, dega2, degb2,
                        b[i].reshape(1, D), x_p, gamma[i].reshape(1, D),
                        beta[i].reshape(1, D), W[i])

    return x_p[:N]
